# Initial kernel scaffold; baseline (speedup 1.0000x reference)
import jax
import jax.numpy as jnp
from jax.experimental import pallas as pl


def kernel(inputs, weight):
    raise NotImplementedError("write your pallas kernel here")


# speedup vs baseline: 2.7161x; 2.7161x over previous
import functools

import jax
import jax.numpy as jnp
from jax import lax
from jax.experimental import pallas as pl
from jax.experimental.pallas import tpu as pltpu
from jax.experimental.pallas import tpu_sc as plsc

NC = 2
NS = 16
NW = NC * NS
LANES = 16

BATCH = 16384
LABELS = 50
EMBED = 64

BW = BATCH // NW
CHUNK = 128
SUB = BW // CHUNK
NCHUNK = SUB * LABELS
NBUF = 4


def _sc_body(idx_hbm, w_hbm, out_hbm, idx_v, acc_v,
             b0, b1, b2, b3, s0, s1, s2, s3):
  bufs = (b0, b1, b2, b3)
  sems = (s0, s1, s2, s3)

  wid = lax.axis_index("s") * NC + lax.axis_index("c")

  pltpu.sync_copy(idx_hbm.at[wid], idx_v)

  for b in range(NBUF):
    pltpu.async_copy(w_hbm.at[idx_v.at[b]], bufs[b], sems[b])

  zero = jnp.zeros((LANES,), jnp.float32)

  @pl.loop(0, BW, unroll=4)
  def _zero(r):
    for c in range(EMBED // LANES):
      acc_v[r, pl.ds(c * LANES, LANES)] = zero

  @pl.loop(0, NCHUNK, step=NBUF)
  def _main(t):
    for b in range(NBUF):
      tt = t + b
      buf = bufs[b]
      sem = sems[b]
      pltpu.make_async_copy(w_hbm.at[idx_v.at[tt]], buf, sem).wait()

      base = (tt % SUB) * CHUNK
      acc_s = acc_v.at[pl.ds(base, CHUNK)]

      @pl.loop(0, CHUNK, unroll=8)
      def _accum(i):
        for c in range(EMBED // LANES):
          v = buf[i, pl.ds(c * LANES, LANES)]
          plsc.addupdate(acc_s.at[i, pl.ds(c * LANES, LANES)], v)

      nxt = tt + NBUF

      @pl.when(nxt < NCHUNK)
      def _():
        pltpu.async_copy(w_hbm.at[idx_v.at[nxt]], buf, sem)

  pltpu.sync_copy(acc_v, out_hbm.at[pl.ds(wid * BW, BW)])


@jax.jit
def _run(idx_r, weight):
  mesh = plsc.VectorSubcoreMesh(
      core_axis_name="c", subcore_axis_name="s",
      num_cores=NC, num_subcores=NS)
  f = pl.kernel(
      _sc_body,
      out_type=jax.ShapeDtypeStruct((BATCH, EMBED), jnp.float32),
      mesh=mesh,
      scratch_types=[
          pltpu.VMEM((NCHUNK, CHUNK), jnp.int32),
          pltpu.VMEM((BW, EMBED), jnp.float32),
      ] + [pltpu.VMEM((CHUNK, EMBED), jnp.float32)] * NBUF
        + [pltpu.SemaphoreType.DMA] * NBUF,
      compiler_params=pltpu.CompilerParams(use_tc_tiling_on_sc=False),
  )
  return f(idx_r, weight)


def kernel(inputs, weight):
  idx = inputs.astype(jnp.int32)
  idx_r = idx.reshape(NW, SUB, CHUNK, LABELS).transpose(0, 3, 1, 2)
  idx_r = idx_r.reshape(NW, NCHUNK, CHUNK)
  return _run(idx_r, weight)
